# baseline (speedup 1.0000x reference)
import functools

import jax
import jax.numpy as jnp
from jax import lax
from jax.experimental import pallas as pl
from jax.experimental.pallas import tpu as pltpu
from jax.experimental.pallas import tpu_sc as plsc

Nt, Nc, Nm = 100000, 20000, 5000
IN, H = 128, 16
NTP = 102400
MOFF = 20800
NACC = 27200
TRASH = 25800
NW = 32
PER_TILE = NTP // NW
RCH = 640
NCHUNK = PER_TILE // RCH
IDXR = PER_TILE // 128
ACC_PER_TILE = NACC // 16


def _mm_wide_body(x_ref, w_ref, b_ref, *o_refs):
    for t in range(8):
        xt = x_ref[:, t, :]
        res = jnp.dot(xt, w_ref[...], preferred_element_type=jnp.float32)
        res = res + b_ref[...]
        for k, o in enumerate(o_refs):
            o[:, 16 * t:16 * (t + 1)] = res[:, 16 * k:16 * (k + 1)]


def _mm_wide(x3, w, b, n_out):
    rows8 = x3.shape[0]
    grid = rows8 // 200
    return pl.pallas_call(
        _mm_wide_body,
        grid=(grid,),
        in_specs=[
            pl.BlockSpec((200, 8, 128), lambda i: (i, 0, 0)),
            pl.BlockSpec((128, 16 * n_out), lambda i: (0, 0)),
            pl.BlockSpec((1, 16 * n_out), lambda i: (0, 0)),
        ],
        out_specs=[pl.BlockSpec((200, 128), lambda i: (i, 0))] * n_out,
        out_shape=[jax.ShapeDtypeStruct((rows8, 128), jnp.float32)] * n_out,
    )(x3, w, b)


def _mm_wide_sel_body(x_ref, w0_ref, w1_ref, b0_ref, b1_ref, o_ref):
    card = pl.program_id(0) < 13
    w = jnp.where(card, w0_ref[...], w1_ref[...])
    b = jnp.where(card, b0_ref[...], b1_ref[...])
    for t in range(8):
        xt = x_ref[:, t, :]
        res = jnp.dot(xt, w, preferred_element_type=jnp.float32) + b
        o_ref[:, 16 * t:16 * (t + 1)] = res


def _mm_node(x3, w0, w1, b0, b1):
    return pl.pallas_call(
        _mm_wide_sel_body,
        grid=(NACC // 1600,),
        in_specs=[
            pl.BlockSpec((200, 8, 128), lambda i: (i, 0, 0)),
            pl.BlockSpec((128, 16), lambda i: (0, 0)),
            pl.BlockSpec((128, 16), lambda i: (0, 0)),
            pl.BlockSpec((1, 16), lambda i: (0, 0)),
            pl.BlockSpec((1, 16), lambda i: (0, 0)),
        ],
        out_specs=pl.BlockSpec((200, 128), lambda i: (i, 0)),
        out_shape=jax.ShapeDtypeStruct((NACC // 8, 128), jnp.float32),
    )(x3, w0, w1, b0, b1)


def _mm_packed_body(x_ref, w_ref, b_ref, *o_refs, act):
    x = x_ref[...]
    if act:
        x = jnp.where(x > 0, x, 0.01 * x)
    for t in range(8):
        xt = x[:, 16 * t:16 * (t + 1)]
        res = jnp.dot(xt, w_ref[...], preferred_element_type=jnp.float32)
        res = res + b_ref[...]
        for k, o in enumerate(o_refs):
            o[:, 16 * t:16 * (t + 1)] = res[:, 16 * k:16 * (k + 1)]


def _mm_packed(xp, w, b, n_out, act):
    return pl.pallas_call(
        functools.partial(_mm_packed_body, act=act),
        grid=(NTP // 1600,),
        in_specs=[
            pl.BlockSpec((200, 128), lambda i: (i, 0)),
            pl.BlockSpec((16, 16 * n_out), lambda i: (0, 0)),
            pl.BlockSpec((1, 16 * n_out), lambda i: (0, 0)),
        ],
        out_specs=[pl.BlockSpec((200, 128), lambda i: (i, 0))] * n_out,
        out_shape=[jax.ShapeDtypeStruct((NTP // 8, 128), jnp.float32)] * n_out,
    )(xp, w, b)


def _agg_node_body(ps_ref, pc_ref, w0_ref, w1_ref, b0_ref, b1_ref,
                   o_ref, cnt_ref, *, emit_cnt):
    if emit_cnt:
        cnt = pc_ref[0] + pc_ref[1]
        cnt_ref[...] = cnt
    else:
        cnt = pc_ref[0]
    x = (ps_ref[0] + ps_ref[1]) / jnp.maximum(cnt, 1.0)
    x = jnp.where(x > 0, x, 0.01 * x)
    card = pl.program_id(0) < 13
    w = jnp.where(card, w0_ref[...], w1_ref[...])
    b = jnp.where(card, b0_ref[...], b1_ref[...])
    for t in range(8):
        xt = x[:, 16 * t:16 * (t + 1)]
        res = jnp.dot(xt, w, preferred_element_type=jnp.float32) + b
        o_ref[:, 16 * t:16 * (t + 1)] = res


def _agg_node(ps, pc, w0, w1, b0, b1, emit_cnt):
    n_out = 2 if emit_cnt else 1
    pspec = pl.BlockSpec((2, 200, 128), lambda i: (0, i, 0))
    cspec = pl.BlockSpec((pc.shape[0], 200, 128), lambda i: (0, i, 0))
    nspec = pl.BlockSpec((200, 128), lambda i: (i, 0))
    wspec = pl.BlockSpec((16, 16), lambda i: (0, 0))
    bspec = pl.BlockSpec((1, 16), lambda i: (0, 0))
    body = functools.partial(_agg_node_body, emit_cnt=emit_cnt)
    if not emit_cnt:
        body = lambda psr, pcr, w0r, w1r, b0r, b1r, o: _agg_node_body(
            psr, pcr, w0r, w1r, b0r, b1r, o, None, emit_cnt=False)
    return pl.pallas_call(
        body,
        grid=(NACC // 1600,),
        in_specs=[pspec, cspec, wspec, wspec, bspec, bspec],
        out_specs=[nspec] * n_out,
        out_shape=[jax.ShapeDtypeStruct((NACC // 8, 128), jnp.float32)] * n_out,
    )(ps, pc, w0, w1, b0, b1)


def _make_sc_pass(do_scatter, do_counts):
    mesh = plsc.VectorSubcoreMesh(core_axis_name="c", subcore_axis_name="s")
    out_type = [jax.ShapeDtypeStruct((NTP, 16), jnp.float32)]
    if do_scatter:
        out_type.append(jax.ShapeDtypeStruct((2, NACC, 16), jnp.float32))
    if do_counts:
        out_type.append(jax.ShapeDtypeStruct((2, NACC, 16), jnp.float32))
    scratch = [
        pltpu.VMEM((PER_TILE,), jnp.int32),
        pltpu.VMEM((PER_TILE,), jnp.int32),
        pltpu.VMEM((RCH, 16), jnp.float32),
        pltpu.VMEM((RCH, 16), jnp.float32),
        pltpu.VMEM((RCH, 16), jnp.float32),
        pltpu.SemaphoreType.DMA,
    ]
    if do_scatter:
        scratch = [
            pltpu.VMEM((IDXR, 128), jnp.int32),
            pltpu.VMEM((IDXR, 128), jnp.int32),
            pltpu.VMEM((RCH, 16), jnp.float32),
            pltpu.VMEM((RCH, 16), jnp.float32),
            pltpu.VMEM((425, 16), jnp.float32),
            pltpu.VMEM_SHARED((NACC, 16), jnp.float32),
        ] + scratch
    if do_counts:
        scratch = [pltpu.VMEM_SHARED((NACC, 16), jnp.float32)] + scratch

    def body(*refs):
        it = iter(refs)
        tself = next(it)
        if do_scatter:
            ttc, ttm = next(it), next(it)
        cm = next(it)
        ia_c, ia_m = next(it), next(it)
        if do_scatter:
            ib_c, ib_m = next(it), next(it)
        x1_o = next(it)
        if do_scatter:
            ps_o = next(it)
        if do_counts:
            pc_o = next(it)
            acc_c = next(it)
        if do_scatter:
            ibc_v, ibm_v, tc_v, tm_v, zb_v, acc_s = (next(it) for _ in range(6))
        iac_v, iam_v, ts_v, gc_v, gm_v, sem = (next(it) for _ in range(6))

        c = lax.axis_index("c")
        s = lax.axis_index("s")
        wid = c * 16 + s

        pltpu.sync_copy(ia_c.at[pl.ds(wid * PER_TILE, PER_TILE)], iac_v)
        pltpu.sync_copy(ia_m.at[pl.ds(wid * PER_TILE, PER_TILE)], iam_v)
        if do_scatter:
            pltpu.sync_copy(ib_c.at[wid], ibc_v)
            pltpu.sync_copy(ib_m.at[wid], ibm_v)

            def zrow(i, _):
                zb_v[i, :] = jnp.zeros((16,), jnp.float32)
                return 0
            lax.fori_loop(0, 425, zrow, 0)
            for k in range(4):
                dst = pl.ds(s * ACC_PER_TILE + k * 425, 425)
                pltpu.sync_copy(zb_v, acc_s.at[dst])
                if do_counts:
                    pltpu.sync_copy(zb_v, acc_c.at[dst])
            if do_counts:
                def orow(i, _):
                    zb_v[i, :] = jnp.ones((16,), jnp.float32)
                    return 0
                lax.fori_loop(0, 425, orow, 0)
            plsc.subcore_barrier()

        for j in range(NCHUNK):
            base = wid * PER_TILE + j * RCH
            pltpu.sync_copy(tself.at[pl.ds(base, RCH)], ts_v)
            if do_scatter:
                pltpu.sync_copy(ttc.at[pl.ds(base, RCH)], tc_v)
                pltpu.sync_copy(ttm.at[pl.ds(base, RCH)], tm_v)
            cps = []
            for jj in range(RCH // 128):
                r = j * RCH + jj * 128
                sl = pl.ds(jj * 128, 128)
                isl = pl.ds(r, 128)
                cps.append(pltpu.async_copy(cm.at[iac_v.at[isl]], gc_v.at[sl], sem))
                cps.append(pltpu.async_copy(cm.at[iam_v.at[isl]], gm_v.at[sl], sem))
            for cp in cps:
                cp.wait()

            def addrow(i, _):
                gc_v[i, :] = gc_v[i, :] + gm_v[i, :] + ts_v[i, :]
                return 0
            lax.fori_loop(0, RCH, addrow, 0, unroll=4)
            pltpu.sync_copy(gc_v, x1_o.at[pl.ds(base, RCH)])

            if do_scatter:
                for jj in range(RCH // 128):
                    r = j * (RCH // 128) + jj
                    sl = pl.ds(jj * 128, 128)
                    pltpu.sync_copy(tc_v.at[sl], acc_s.at[ibc_v.at[r]], add=True)
                    pltpu.sync_copy(tm_v.at[sl], acc_s.at[ibm_v.at[r]], add=True)
                    if do_counts:
                        one_sl = pl.ds(0, 128)
                        pltpu.sync_copy(zb_v.at[one_sl], acc_c.at[ibc_v.at[r]], add=True)
                        pltpu.sync_copy(zb_v.at[one_sl], acc_c.at[ibm_v.at[r]], add=True)

        if do_scatter:
            plsc.subcore_barrier()
            for k in range(2):
                sl = pl.ds(s * ACC_PER_TILE + k * 850, 850)
                pltpu.sync_copy(acc_s.at[sl], ps_o.at[c, sl])
                if do_counts:
                    pltpu.sync_copy(acc_c.at[sl], pc_o.at[c, sl])

    return pl.kernel(body, out_type=out_type, mesh=mesh,
                     scratch_types=scratch,
                     compiler_params=pltpu.CompilerParams(
                         use_tc_tiling_on_sc=False))


def kernel(features, card_idx, merchant_idx, params):
    prm = params
    L = prm['layers']

    def lw(i, name):
        w, b = L[i][name]
        return w, b.reshape(1, -1)

    w_ct0, b_ct0 = lw(0, 'card_id<>transaction')
    w_mt0, b_mt0 = lw(0, 'merchant_id<>transaction')
    w_ss0, b_ss0 = lw(0, 'self_relation')
    w_tc0, b_tc0 = lw(0, 'transaction<>card_id')
    w_tm0, b_tm0 = lw(0, 'transaction<>merchant_id')
    w_ct1, b_ct1 = lw(1, 'card_id<>transaction')
    w_mt1, b_mt1 = lw(1, 'merchant_id<>transaction')
    w_ss1, b_ss1 = lw(1, 'self_relation')
    w_tc1, b_tc1 = lw(1, 'transaction<>card_id')
    w_tm1, b_tm1 = lw(1, 'transaction<>merchant_id')
    w_ct2, b_ct2 = lw(2, 'card_id<>transaction')
    w_mt2, b_mt2 = lw(2, 'merchant_id<>transaction')
    w_ss2, b_ss2 = lw(2, 'self_relation')
    linw, linb = prm['lin_W'], prm['lin_b']

    wcat0 = jnp.concatenate([w_ss0, w_tc0, w_tm0], axis=1)
    bcat0 = jnp.concatenate([b_ss0, b_tc0, b_tm0], axis=1)
    wcat1 = jnp.concatenate([w_ss1, w_tc1, w_tm1], axis=1)
    bcat1 = jnp.concatenate([b_ss1, b_tc1, b_tm1], axis=1)

    linw_p = jnp.pad(linw, ((0, 0), (0, 16 - linw.shape[1])))
    linb_p = jnp.pad(linb.reshape(1, -1), ((0, 0), (0, 16 - linw.shape[1])))

    ep = jnp.concatenate([
        prm['embed_card'],
        jnp.zeros((MOFF - Nc, IN), jnp.float32),
        prm['embed_merchant'],
        jnp.zeros((NACC - MOFF - Nm, IN), jnp.float32)])

    ci = card_idx.astype(jnp.int32)
    mo = merchant_idx.astype(jnp.int32) + MOFF
    pad = NTP - Nt
    ia_c = jnp.pad(ci, (0, pad))
    ia_m = jnp.pad(mo, (0, pad))
    ib_c = jnp.pad(ci, (0, pad), constant_values=TRASH).reshape(NW, IDXR, 128)
    ib_m = jnp.pad(mo, (0, pad), constant_values=TRASH).reshape(NW, IDXR, 128)

    def to16(a):
        return a.reshape(a.shape[0] * 8, 16)

    feats3 = jnp.pad(features, ((0, pad), (0, 0))).reshape(NTP // 8, 8, 128)
    tself, ttc, ttm = _mm_wide(feats3, wcat0, bcat0, 3)
    cm0 = _mm_node(ep.reshape(NACC // 8, 8, 128), w_ct0, w_mt0, b_ct0, b_mt0)
    x1, ps1, pc1 = _make_sc_pass(True, True)(
        to16(tself), to16(ttc), to16(ttm), to16(cm0),
        ia_c, ia_m, ib_c, ib_m)

    us, utc, utm = _mm_packed(x1.reshape(NTP // 8, 128), wcat1, bcat1, 3, True)
    cm1, cnt = _agg_node(ps1.reshape(2, NACC // 8, 128),
                         pc1.reshape(2, NACC // 8, 128),
                         w_ct1, w_mt1, b_ct1, b_mt1, True)
    x2, ps2 = _make_sc_pass(True, False)(
        to16(us), to16(utc), to16(utm), to16(cm1), ia_c, ia_m, ib_c, ib_m)

    (t2,) = _mm_packed(x2.reshape(NTP // 8, 128), w_ss2, b_ss2, 1, True)
    (cm2,) = _agg_node(ps2.reshape(2, NACC // 8, 128), cnt[None],
                       w_ct2, w_mt2, b_ct2, b_mt2, False)
    (x3,) = _make_sc_pass(False, False)(to16(t2), to16(cm2), ia_c, ia_m)
    (out,) = _mm_packed(x3.reshape(NTP // 8, 128), linw_p, linb_p, 1, False)
    return to16(out)[:Nt, :linw.shape[1]]


# speedup vs baseline: 1.1005x; 1.1005x over previous
import functools

import jax
import jax.numpy as jnp
from jax import lax
from jax.experimental import pallas as pl
from jax.experimental.pallas import tpu as pltpu
from jax.experimental.pallas import tpu_sc as plsc

Nt, Nc, Nm = 100000, 20000, 5000
IN, H = 128, 16
NTP = 102400
MOFF = 20800
NACC = 27200
TRASH = 25800
NW = 32
PER_TILE = NTP // NW
RCH = 640
NCHUNK = PER_TILE // RCH
IDXR = PER_TILE // 128
ACC_PER_TILE = NACC // 16


def _mm_wide_body(x_ref, w_ref, b_ref, *o_refs):
    for t in range(8):
        xt = x_ref[:, t, :]
        res = jnp.dot(xt, w_ref[...], preferred_element_type=jnp.float32)
        res = res + b_ref[...]
        for k, o in enumerate(o_refs):
            o[:, 16 * t:16 * (t + 1)] = res[:, 16 * k:16 * (k + 1)]


def _mm_wide(x3, w, b, n_out):
    rows8 = x3.shape[0]
    grid = rows8 // 200
    return pl.pallas_call(
        _mm_wide_body,
        grid=(grid,),
        in_specs=[
            pl.BlockSpec((200, 8, 128), lambda i: (i, 0, 0)),
            pl.BlockSpec((128, 16 * n_out), lambda i: (0, 0)),
            pl.BlockSpec((1, 16 * n_out), lambda i: (0, 0)),
        ],
        out_specs=[pl.BlockSpec((200, 128), lambda i: (i, 0))] * n_out,
        out_shape=[jax.ShapeDtypeStruct((rows8, 128), jnp.float32)] * n_out,
    )(x3, w, b)


def _mm_wide_sel_body(x_ref, w0_ref, w1_ref, b0_ref, b1_ref, o_ref):
    card = pl.program_id(0) < 13
    w = jnp.where(card, w0_ref[...], w1_ref[...])
    b = jnp.where(card, b0_ref[...], b1_ref[...])
    for t in range(8):
        xt = x_ref[:, t, :]
        res = jnp.dot(xt, w, preferred_element_type=jnp.float32) + b
        o_ref[:, 16 * t:16 * (t + 1)] = res


def _mm_node(x3, w0, w1, b0, b1):
    return pl.pallas_call(
        _mm_wide_sel_body,
        grid=(NACC // 1600,),
        in_specs=[
            pl.BlockSpec((200, 8, 128), lambda i: (i, 0, 0)),
            pl.BlockSpec((128, 16), lambda i: (0, 0)),
            pl.BlockSpec((128, 16), lambda i: (0, 0)),
            pl.BlockSpec((1, 16), lambda i: (0, 0)),
            pl.BlockSpec((1, 16), lambda i: (0, 0)),
        ],
        out_specs=pl.BlockSpec((200, 128), lambda i: (i, 0)),
        out_shape=jax.ShapeDtypeStruct((NACC // 8, 128), jnp.float32),
    )(x3, w0, w1, b0, b1)


def _blockdiag(w, b):
    n = w.shape[1] // 16
    wbig = jnp.concatenate(
        [jnp.kron(jnp.eye(8, dtype=w.dtype), w[:, 16 * o:16 * (o + 1)])
         for o in range(n)], axis=1)
    bbig = jnp.concatenate(
        [jnp.tile(b[:, 16 * o:16 * (o + 1)], (1, 8)) for o in range(n)],
        axis=1)
    return wbig, bbig


def _mm_packed_body(x_ref, w_ref, b_ref, *o_refs, act):
    x = x_ref[...]
    if act:
        x = jnp.where(x > 0, x, 0.01 * x)
    res = jnp.dot(x, w_ref[...], preferred_element_type=jnp.float32)
    res = res + b_ref[...]
    for k, o in enumerate(o_refs):
        o[...] = res[:, 128 * k:128 * (k + 1)]


def _mm_packed(xp, wbig, bbig, n_out, act):
    return pl.pallas_call(
        functools.partial(_mm_packed_body, act=act),
        grid=(NTP // 1600,),
        in_specs=[
            pl.BlockSpec((200, 128), lambda i: (i, 0)),
            pl.BlockSpec((128, 128 * n_out), lambda i: (0, 0)),
            pl.BlockSpec((1, 128 * n_out), lambda i: (0, 0)),
        ],
        out_specs=[pl.BlockSpec((200, 128), lambda i: (i, 0))] * n_out,
        out_shape=[jax.ShapeDtypeStruct((NTP // 8, 128), jnp.float32)] * n_out,
    )(xp, wbig, bbig)


def _agg_node_body(ps_ref, pc_ref, w0_ref, b0_ref, w1_ref, b1_ref,
                   o_ref, cnt_ref, *, emit_cnt):
    if emit_cnt:
        cnt = pc_ref[0] + pc_ref[1]
        cnt_ref[...] = cnt
    else:
        cnt = pc_ref[0]
    x = (ps_ref[0] + ps_ref[1]) / jnp.maximum(cnt, 1.0)
    x = jnp.where(x > 0, x, 0.01 * x)
    card = pl.program_id(0) < 13
    w = jnp.where(card, w0_ref[...], w1_ref[...])
    b = jnp.where(card, b0_ref[...], b1_ref[...])
    o_ref[...] = jnp.dot(x, w, preferred_element_type=jnp.float32) + b


def _agg_node(ps, pc, w0, b0, w1, b1, emit_cnt):
    n_out = 2 if emit_cnt else 1
    pspec = pl.BlockSpec((2, 200, 128), lambda i: (0, i, 0))
    cspec = pl.BlockSpec((pc.shape[0], 200, 128), lambda i: (0, i, 0))
    nspec = pl.BlockSpec((200, 128), lambda i: (i, 0))
    wspec = pl.BlockSpec((128, 128), lambda i: (0, 0))
    bspec = pl.BlockSpec((1, 128), lambda i: (0, 0))
    body = functools.partial(_agg_node_body, emit_cnt=emit_cnt)
    if not emit_cnt:
        body = lambda psr, pcr, w0r, b0r, w1r, b1r, o: _agg_node_body(
            psr, pcr, w0r, b0r, w1r, b1r, o, None, emit_cnt=False)
    return pl.pallas_call(
        body,
        grid=(NACC // 1600,),
        in_specs=[pspec, cspec, wspec, bspec, wspec, bspec],
        out_specs=[nspec] * n_out,
        out_shape=[jax.ShapeDtypeStruct((NACC // 8, 128), jnp.float32)] * n_out,
    )(ps, pc, w0, b0, w1, b1)


def _make_sc_pass(do_scatter, do_counts):
    mesh = plsc.VectorSubcoreMesh(core_axis_name="c", subcore_axis_name="s")
    out_type = [jax.ShapeDtypeStruct((NTP, 16), jnp.float32)]
    if do_scatter:
        out_type.append(jax.ShapeDtypeStruct((2, NACC, 16), jnp.float32))
    if do_counts:
        out_type.append(jax.ShapeDtypeStruct((2, NACC, 16), jnp.float32))
    scratch = [
        pltpu.VMEM((PER_TILE,), jnp.int32),
        pltpu.VMEM((PER_TILE,), jnp.int32),
        pltpu.VMEM((RCH, 16), jnp.float32),
        pltpu.VMEM((RCH, 16), jnp.float32),
        pltpu.VMEM((RCH, 16), jnp.float32),
        pltpu.SemaphoreType.DMA,
    ]
    if do_scatter:
        scratch = [
            pltpu.VMEM((IDXR, 128), jnp.int32),
            pltpu.VMEM((IDXR, 128), jnp.int32),
            pltpu.VMEM((RCH, 16), jnp.float32),
            pltpu.VMEM((RCH, 16), jnp.float32),
            pltpu.VMEM((425, 16), jnp.float32),
            pltpu.VMEM_SHARED((NACC, 16), jnp.float32),
        ] + scratch
    if do_counts:
        scratch = [pltpu.VMEM_SHARED((NACC, 16), jnp.float32)] + scratch

    def body(*refs):
        it = iter(refs)
        tself = next(it)
        if do_scatter:
            ttc, ttm = next(it), next(it)
        cm = next(it)
        ia_c, ia_m = next(it), next(it)
        if do_scatter:
            ib_c, ib_m = next(it), next(it)
        x1_o = next(it)
        if do_scatter:
            ps_o = next(it)
        if do_counts:
            pc_o = next(it)
            acc_c = next(it)
        if do_scatter:
            ibc_v, ibm_v, tc_v, tm_v, zb_v, acc_s = (next(it) for _ in range(6))
        iac_v, iam_v, ts_v, gc_v, gm_v, sem = (next(it) for _ in range(6))

        c = lax.axis_index("c")
        s = lax.axis_index("s")
        wid = c * 16 + s

        pltpu.sync_copy(ia_c.at[pl.ds(wid * PER_TILE, PER_TILE)], iac_v)
        pltpu.sync_copy(ia_m.at[pl.ds(wid * PER_TILE, PER_TILE)], iam_v)
        if do_scatter:
            pltpu.sync_copy(ib_c.at[wid], ibc_v)
            pltpu.sync_copy(ib_m.at[wid], ibm_v)

            def zrow(i, _):
                zb_v[i, :] = jnp.zeros((16,), jnp.float32)
                return 0
            lax.fori_loop(0, 425, zrow, 0)
            for k in range(4):
                dst = pl.ds(s * ACC_PER_TILE + k * 425, 425)
                pltpu.sync_copy(zb_v, acc_s.at[dst])
                if do_counts:
                    pltpu.sync_copy(zb_v, acc_c.at[dst])
            if do_counts:
                def orow(i, _):
                    zb_v[i, :] = jnp.ones((16,), jnp.float32)
                    return 0
                lax.fori_loop(0, 425, orow, 0)
            plsc.subcore_barrier()

        for j in range(NCHUNK):
            base = wid * PER_TILE + j * RCH
            pltpu.sync_copy(tself.at[pl.ds(base, RCH)], ts_v)
            if do_scatter:
                pltpu.sync_copy(ttc.at[pl.ds(base, RCH)], tc_v)
                pltpu.sync_copy(ttm.at[pl.ds(base, RCH)], tm_v)
            cps = []
            for jj in range(RCH // 128):
                r = j * RCH + jj * 128
                sl = pl.ds(jj * 128, 128)
                isl = pl.ds(r, 128)
                cps.append(pltpu.async_copy(cm.at[iac_v.at[isl]], gc_v.at[sl], sem))
                cps.append(pltpu.async_copy(cm.at[iam_v.at[isl]], gm_v.at[sl], sem))
            for cp in cps:
                cp.wait()

            def addrow(i, _):
                gc_v[i, :] = gc_v[i, :] + gm_v[i, :] + ts_v[i, :]
                return 0
            lax.fori_loop(0, RCH, addrow, 0, unroll=4)
            pltpu.sync_copy(gc_v, x1_o.at[pl.ds(base, RCH)])

            if do_scatter:
                for jj in range(RCH // 128):
                    r = j * (RCH // 128) + jj
                    sl = pl.ds(jj * 128, 128)
                    pltpu.sync_copy(tc_v.at[sl], acc_s.at[ibc_v.at[r]], add=True)
                    pltpu.sync_copy(tm_v.at[sl], acc_s.at[ibm_v.at[r]], add=True)
                    if do_counts:
                        one_sl = pl.ds(0, 128)
                        pltpu.sync_copy(zb_v.at[one_sl], acc_c.at[ibc_v.at[r]], add=True)
                        pltpu.sync_copy(zb_v.at[one_sl], acc_c.at[ibm_v.at[r]], add=True)

        if do_scatter:
            plsc.subcore_barrier()
            for k in range(2):
                sl = pl.ds(s * ACC_PER_TILE + k * 850, 850)
                pltpu.sync_copy(acc_s.at[sl], ps_o.at[c, sl])
                if do_counts:
                    pltpu.sync_copy(acc_c.at[sl], pc_o.at[c, sl])

    return pl.kernel(body, out_type=out_type, mesh=mesh,
                     scratch_types=scratch,
                     compiler_params=pltpu.CompilerParams(
                         use_tc_tiling_on_sc=False))


def kernel(features, card_idx, merchant_idx, params):
    prm = params
    L = prm['layers']

    def lw(i, name):
        w, b = L[i][name]
        return w, b.reshape(1, -1)

    w_ct0, b_ct0 = lw(0, 'card_id<>transaction')
    w_mt0, b_mt0 = lw(0, 'merchant_id<>transaction')
    w_ss0, b_ss0 = lw(0, 'self_relation')
    w_tc0, b_tc0 = lw(0, 'transaction<>card_id')
    w_tm0, b_tm0 = lw(0, 'transaction<>merchant_id')
    w_ct1, b_ct1 = lw(1, 'card_id<>transaction')
    w_mt1, b_mt1 = lw(1, 'merchant_id<>transaction')
    w_ss1, b_ss1 = lw(1, 'self_relation')
    w_tc1, b_tc1 = lw(1, 'transaction<>card_id')
    w_tm1, b_tm1 = lw(1, 'transaction<>merchant_id')
    w_ct2, b_ct2 = lw(2, 'card_id<>transaction')
    w_mt2, b_mt2 = lw(2, 'merchant_id<>transaction')
    w_ss2, b_ss2 = lw(2, 'self_relation')
    linw, linb = prm['lin_W'], prm['lin_b']

    wcat0 = jnp.concatenate([w_ss0, w_tc0, w_tm0], axis=1)
    bcat0 = jnp.concatenate([b_ss0, b_tc0, b_tm0], axis=1)
    wcat1 = jnp.concatenate([w_ss1, w_tc1, w_tm1], axis=1)
    bcat1 = jnp.concatenate([b_ss1, b_tc1, b_tm1], axis=1)

    linw_p = jnp.pad(linw, ((0, 0), (0, 16 - linw.shape[1])))
    linb_p = jnp.pad(linb.reshape(1, -1), ((0, 0), (0, 16 - linw.shape[1])))

    ep = jnp.concatenate([
        prm['embed_card'],
        jnp.zeros((MOFF - Nc, IN), jnp.float32),
        prm['embed_merchant'],
        jnp.zeros((NACC - MOFF - Nm, IN), jnp.float32)])

    ci = card_idx.astype(jnp.int32)
    mo = merchant_idx.astype(jnp.int32) + MOFF
    pad = NTP - Nt
    ia_c = jnp.pad(ci, (0, pad))
    ia_m = jnp.pad(mo, (0, pad))
    ib_c = jnp.pad(ci, (0, pad), constant_values=TRASH).reshape(NW, IDXR, 128)
    ib_m = jnp.pad(mo, (0, pad), constant_values=TRASH).reshape(NW, IDXR, 128)

    def to16(a):
        return a.reshape(a.shape[0] * 8, 16)

    feats3 = jnp.pad(features, ((0, pad), (0, 0))).reshape(NTP // 8, 8, 128)
    tself, ttc, ttm = _mm_wide(feats3, wcat0, bcat0, 3)
    cm0 = _mm_node(ep.reshape(NACC // 8, 8, 128), w_ct0, w_mt0, b_ct0, b_mt0)
    x1, ps1, pc1 = _make_sc_pass(True, True)(
        to16(tself), to16(ttc), to16(ttm), to16(cm0),
        ia_c, ia_m, ib_c, ib_m)

    us, utc, utm = _mm_packed(x1.reshape(NTP // 8, 128),
                              *_blockdiag(wcat1, bcat1), 3, True)
    cm1, cnt = _agg_node(ps1.reshape(2, NACC // 8, 128),
                         pc1.reshape(2, NACC // 8, 128),
                         *_blockdiag(w_ct1, b_ct1),
                         *_blockdiag(w_mt1, b_mt1), True)
    x2, ps2 = _make_sc_pass(True, False)(
        to16(us), to16(utc), to16(utm), to16(cm1), ia_c, ia_m, ib_c, ib_m)

    (t2,) = _mm_packed(x2.reshape(NTP // 8, 128),
                       *_blockdiag(w_ss2, b_ss2), 1, True)
    (cm2,) = _agg_node(ps2.reshape(2, NACC // 8, 128), cnt[None],
                       *_blockdiag(w_ct2, b_ct2),
                       *_blockdiag(w_mt2, b_mt2), False)
    (x3,) = _make_sc_pass(False, False)(to16(t2), to16(cm2), ia_c, ia_m)
    (out,) = _mm_packed(x3.reshape(NTP // 8, 128),
                        *_blockdiag(linw_p, linb_p), 1, False)
    return to16(out)[:Nt, :linw.shape[1]]
